# baseline (device time: 152228 ns/iter reference)
import jax
import jax.numpy as jnp
from jax import lax
from jax.experimental import pallas as pl
from jax.experimental.pallas import tpu as pltpu

N_ROWS = 2048
N_COLS = 1024
CHUNK = 256
MAX_CHUNKS = N_ROWS // CHUNK


def kernel(x, dest):
    iota = jnp.arange(N_ROWS, dtype=jnp.int32)
    keys = jnp.sort(dest * N_ROWS + iota)
    order = keys & (N_ROWS - 1)
    k0 = (N_ROWS - jnp.sum(dest)).astype(jnp.int32)
    xs = jnp.take(x, jnp.roll(order, -k0), axis=0)
    info = jnp.reshape(k0, (1,))

    def body(info_ref, x_ref, out_ref, send_ref, kept_ref, recv_ref,
             send_sems, recv_sems):
        my_x = lax.axis_index("x")
        my_y = lax.axis_index("y")
        my_z = lax.axis_index("z")
        partner = (1 - my_x, my_y, my_z)
        k0 = info_ref[0]
        is0 = my_x == 0

        send_lo = jnp.where(is0, 0, N_ROWS - k0)
        send_hi = jnp.where(is0, N_ROWS - k0, N_ROWS)
        recv_lo = jnp.where(is0, k0, 0)
        recv_hi = jnp.where(is0, N_ROWS, k0)

        barrier_sem = pltpu.get_barrier_semaphore()
        pl.semaphore_signal(
            barrier_sem, inc=1,
            device_id=partner, device_id_type=pl.DeviceIdType.MESH,
        )
        pl.semaphore_wait(barrier_sem, 1)

        for i in range(MAX_CHUNKS):
            sl = pl.ds(i * CHUNK, CHUNK)

            @pl.when((i * CHUNK < send_hi) & ((i + 1) * CHUNK > send_lo))
            def _():
                send_ref[sl] = x_ref[sl].astype(jnp.bfloat16)
                rdma = pltpu.make_async_remote_copy(
                    src_ref=send_ref.at[sl],
                    dst_ref=recv_ref.at[sl],
                    send_sem=send_sems.at[i],
                    recv_sem=recv_sems.at[i],
                    device_id=partner,
                    device_id_type=pl.DeviceIdType.MESH,
                )
                rdma.start()

        kept_ref[...] = pltpu.roll(
            x_ref[...].astype(jnp.bfloat16), k0, axis=0
        )

        for i in range(MAX_CHUNKS):
            recv_here = (i * CHUNK < recv_hi) & ((i + 1) * CHUNK > recv_lo)
            row = i * CHUNK + lax.broadcasted_iota(jnp.int32, (CHUNK, N_COLS), 0)
            sl = pl.ds(i * CHUNK, CHUNK)

            @pl.when(recv_here)
            def _():
                rdma = pltpu.make_async_remote_copy(
                    src_ref=send_ref.at[sl],
                    dst_ref=recv_ref.at[sl],
                    send_sem=send_sems.at[i],
                    recv_sem=recv_sems.at[i],
                    device_id=partner,
                    device_id_type=pl.DeviceIdType.MESH,
                )
                rdma.wait_recv()
                out_ref[sl] = jnp.where(
                    (row < k0) == is0, kept_ref[sl], recv_ref[sl]
                )

            @pl.when(jnp.logical_not(recv_here))
            def _():
                out_ref[sl] = kept_ref[sl]

        for i in range(MAX_CHUNKS):
            sl = pl.ds(i * CHUNK, CHUNK)

            @pl.when((i * CHUNK < send_hi) & ((i + 1) * CHUNK > send_lo))
            def _():
                rdma = pltpu.make_async_remote_copy(
                    src_ref=send_ref.at[sl],
                    dst_ref=recv_ref.at[sl],
                    send_sem=send_sems.at[i],
                    recv_sem=recv_sems.at[i],
                    device_id=partner,
                    device_id_type=pl.DeviceIdType.MESH,
                )
                rdma.wait_send()

    return pl.pallas_call(
        body,
        out_shape=jax.ShapeDtypeStruct((N_ROWS, N_COLS), jnp.bfloat16),
        in_specs=[
            pl.BlockSpec(memory_space=pltpu.SMEM),
            pl.BlockSpec(memory_space=pltpu.VMEM),
        ],
        out_specs=pl.BlockSpec(memory_space=pltpu.VMEM),
        scratch_shapes=[
            pltpu.VMEM((N_ROWS, N_COLS), jnp.bfloat16),
            pltpu.VMEM((N_ROWS, N_COLS), jnp.bfloat16),
            pltpu.VMEM((N_ROWS, N_COLS), jnp.bfloat16),
            pltpu.SemaphoreType.DMA((MAX_CHUNKS,)),
            pltpu.SemaphoreType.DMA((MAX_CHUNKS,)),
        ],
        compiler_params=pltpu.CompilerParams(collective_id=0),
    )(info, xs)


# device time: 43187 ns/iter; 3.5249x vs baseline; 3.5249x over previous
import jax
import jax.numpy as jnp
from jax import lax
from jax.experimental import pallas as pl
from jax.experimental.pallas import tpu as pltpu

N_ROWS = 2048
N_COLS = 1024
CHUNK = 256
MAX_CHUNKS = N_ROWS // CHUNK


def kernel(x, dest):
    iota = jnp.arange(N_ROWS, dtype=jnp.int32)
    keys = jnp.sort(dest * N_ROWS + iota)
    order = keys & (N_ROWS - 1)
    k0 = (N_ROWS - jnp.sum(dest)).astype(jnp.int32)
    xs = jnp.take(x, jnp.roll(order, -k0), axis=0).astype(jnp.bfloat16)
    info = jnp.reshape(k0, (1,))

    def body(info_ref, x_ref, out_ref, kept_ref, recv_ref,
             send_sems, recv_sems):
        my_x = lax.axis_index("x")
        my_y = lax.axis_index("y")
        my_z = lax.axis_index("z")
        partner = (1 - my_x, my_y, my_z)
        k0 = info_ref[0]
        is0 = my_x == 0

        send_lo = jnp.where(is0, 0, N_ROWS - k0)
        send_hi = jnp.where(is0, N_ROWS - k0, N_ROWS)
        recv_lo = jnp.where(is0, k0, 0)
        recv_hi = jnp.where(is0, N_ROWS, k0)

        barrier_sem = pltpu.get_barrier_semaphore()
        pl.semaphore_signal(
            barrier_sem, inc=1,
            device_id=partner, device_id_type=pl.DeviceIdType.MESH,
        )
        pl.semaphore_wait(barrier_sem, 1)

        for i in range(MAX_CHUNKS):
            sl = pl.ds(i * CHUNK, CHUNK)

            @pl.when((i * CHUNK < send_hi) & ((i + 1) * CHUNK > send_lo))
            def _():
                rdma = pltpu.make_async_remote_copy(
                    src_ref=x_ref.at[sl],
                    dst_ref=recv_ref.at[sl],
                    send_sem=send_sems.at[i],
                    recv_sem=recv_sems.at[i],
                    device_id=partner,
                    device_id_type=pl.DeviceIdType.MESH,
                )
                rdma.start()

        kept_ref[...] = pltpu.roll(x_ref[...], k0, axis=0)

        for i in range(MAX_CHUNKS):
            recv_here = (i * CHUNK < recv_hi) & ((i + 1) * CHUNK > recv_lo)
            row = i * CHUNK + lax.broadcasted_iota(jnp.int32, (CHUNK, N_COLS), 0)
            sl = pl.ds(i * CHUNK, CHUNK)

            @pl.when(recv_here)
            def _():
                rdma = pltpu.make_async_remote_copy(
                    src_ref=x_ref.at[sl],
                    dst_ref=recv_ref.at[sl],
                    send_sem=send_sems.at[i],
                    recv_sem=recv_sems.at[i],
                    device_id=partner,
                    device_id_type=pl.DeviceIdType.MESH,
                )
                rdma.wait_recv()
                out_ref[sl] = jnp.where(
                    (row < k0) == is0, kept_ref[sl], recv_ref[sl]
                )

            @pl.when(jnp.logical_not(recv_here))
            def _():
                out_ref[sl] = kept_ref[sl]

        for i in range(MAX_CHUNKS):
            sl = pl.ds(i * CHUNK, CHUNK)

            @pl.when((i * CHUNK < send_hi) & ((i + 1) * CHUNK > send_lo))
            def _():
                rdma = pltpu.make_async_remote_copy(
                    src_ref=x_ref.at[sl],
                    dst_ref=recv_ref.at[sl],
                    send_sem=send_sems.at[i],
                    recv_sem=recv_sems.at[i],
                    device_id=partner,
                    device_id_type=pl.DeviceIdType.MESH,
                )
                rdma.wait_send()

    return pl.pallas_call(
        body,
        out_shape=jax.ShapeDtypeStruct((N_ROWS, N_COLS), jnp.bfloat16),
        in_specs=[
            pl.BlockSpec(memory_space=pltpu.SMEM),
            pl.BlockSpec(memory_space=pltpu.VMEM),
        ],
        out_specs=pl.BlockSpec(memory_space=pltpu.VMEM),
        scratch_shapes=[
            pltpu.VMEM((N_ROWS, N_COLS), jnp.bfloat16),
            pltpu.VMEM((N_ROWS, N_COLS), jnp.bfloat16),
            pltpu.SemaphoreType.DMA((MAX_CHUNKS,)),
            pltpu.SemaphoreType.DMA((MAX_CHUNKS,)),
        ],
        compiler_params=pltpu.CompilerParams(collective_id=0),
    )(info, xs)
